# baseline (speedup 1.0000x reference)
import jax
import jax.numpy as jnp
from jax.experimental import pallas as pl
from jax.experimental.pallas import tpu as pltpu
from jax.experimental.pallas import tpu_sc as plsc

_BN = 512
_GATHER_WINDOW = 128


def _dist_argmin_body(x_ref, et_ref, x2_ref, e2_ref, iota_ref, idx_ref):
    xb = x_ref[...]
    et = et_ref[...]
    cross = jax.lax.dot_general(
        xb.astype(jnp.bfloat16), et.astype(jnp.bfloat16),
        (((1,), (0,)), ((), ())),
        preferred_element_type=jnp.float32)
    x2 = x2_ref[0, 0, :]
    e2 = e2_ref[0, 0, :]
    dist = (x2[:, None] + e2[None, :]) - 2.0 * cross
    m = jnp.min(dist, axis=1, keepdims=True)
    iota = iota_ref[0, 0, :]
    k = dist.shape[1]
    cand = jnp.where(dist == m, iota[None, :], jnp.float32(k))
    idx_ref[0, :] = jnp.min(cand, axis=1).astype(jnp.int32)


def _argmin_indices(xp, et, x2, e2):
    n, d = xp.shape
    k = et.shape[1]
    nblocks = n // _BN
    iota = jnp.arange(k, dtype=jnp.float32).reshape(1, 1, k)
    idx = pl.pallas_call(
        _dist_argmin_body,
        grid=(nblocks,),
        in_specs=[
            pl.BlockSpec((_BN, d), lambda i: (i, 0)),
            pl.BlockSpec((d, k), lambda i: (0, 0)),
            pl.BlockSpec((1, 1, _BN), lambda i: (i, 0, 0)),
            pl.BlockSpec((1, 1, k), lambda i: (0, 0, 0)),
            pl.BlockSpec((1, 1, k), lambda i: (0, 0, 0)),
        ],
        out_specs=pl.BlockSpec((1, _BN), lambda i: (0, i)),
        out_shape=jax.ShapeDtypeStruct((1, n), jnp.int32),
    )(xp, et, x2.reshape(nblocks, 1, _BN), e2.reshape(1, 1, k), iota)
    return idx


def _sc_gather(table, indices):
    n = indices.shape[1]
    d = table.shape[1]

    @pl.kernel(
        out_type=jax.ShapeDtypeStruct((n, d), table.dtype),
        mesh=plsc.VectorSubcoreMesh(core_axis_name="core",
                                    subcore_axis_name="subcore"),
    )
    def gather_kernel(tab_hbm, i_hbm, o_hbm):
        def body(i_vmem, o_vmem):
            pltpu.sync_copy(tab_hbm.at[i_vmem.at[0]], o_vmem)

        pltpu.emit_pipeline(
            body,
            grid=(n // _GATHER_WINDOW,),
            in_specs=[pl.BlockSpec((1, _GATHER_WINDOW),
                                   index_map=lambda i: (0, i))],
            out_specs=[pl.BlockSpec((_GATHER_WINDOW, d),
                                    index_map=lambda i: (i, 0))],
            core_axis_name=("core", "subcore"),
            dimension_semantics=(pltpu.PARALLEL,),
        )(i_hbm, o_hbm)

    return gather_kernel(table, indices)


def kernel(x, embeddings):
    b, c, h, w = x.shape
    xp = jnp.transpose(x, (0, 2, 3, 1)).reshape(b * h * w, c)
    x2 = jnp.sum(xp * xp, axis=-1)
    e2 = jnp.sum(embeddings * embeddings, axis=-1)
    k, d = embeddings.shape
    table = jnp.pad(embeddings, ((0, 0), (0, 128 - d)))
    et = embeddings.T
    n = b * h * w
    hn = n // 2
    outs = []
    for lo in (0, hn):
        idx_h = _argmin_indices(xp[lo:lo + hn], et, x2[lo:lo + hn], e2)
        xq_h = _sc_gather(table, idx_h)[:, :d]
        outs.append(xq_h)
    xq = jnp.concatenate(outs, axis=0).reshape(b, h, w, c)
    xq = jnp.transpose(xq, (0, 3, 1, 2))
    return jax.lax.stop_gradient(xq - x) + x


# speedup vs baseline: 1.0390x; 1.0390x over previous
import jax
import jax.numpy as jnp
from jax.experimental import pallas as pl
from jax.experimental.pallas import tpu as pltpu
from jax.experimental.pallas import tpu_sc as plsc

_BN = 512
_GATHER_WINDOW = 256


def _dist_argmin_body(x_ref, et_ref, x2_ref, e2_ref, iota_ref, idx_ref):
    xb = x_ref[...]
    et = et_ref[...]
    cross = jax.lax.dot_general(
        xb.astype(jnp.bfloat16), et.astype(jnp.bfloat16),
        (((1,), (0,)), ((), ())),
        preferred_element_type=jnp.float32)
    x2 = x2_ref[0, 0, :]
    e2 = e2_ref[0, 0, :]
    dist = (x2[:, None] + e2[None, :]) - 2.0 * cross
    m = jnp.min(dist, axis=1, keepdims=True)
    iota = iota_ref[0, 0, :]
    k = dist.shape[1]
    cand = jnp.where(dist == m, iota[None, :], jnp.float32(k))
    idx_ref[0, :] = jnp.min(cand, axis=1).astype(jnp.int32)


def _argmin_indices(xp, et, x2, e2):
    n, d = xp.shape
    k = et.shape[1]
    nblocks = n // _BN
    iota = jnp.arange(k, dtype=jnp.float32).reshape(1, 1, k)
    idx = pl.pallas_call(
        _dist_argmin_body,
        grid=(nblocks,),
        in_specs=[
            pl.BlockSpec((_BN, d), lambda i: (i, 0)),
            pl.BlockSpec((d, k), lambda i: (0, 0)),
            pl.BlockSpec((1, 1, _BN), lambda i: (i, 0, 0)),
            pl.BlockSpec((1, 1, k), lambda i: (0, 0, 0)),
            pl.BlockSpec((1, 1, k), lambda i: (0, 0, 0)),
        ],
        out_specs=pl.BlockSpec((1, _BN), lambda i: (0, i)),
        out_shape=jax.ShapeDtypeStruct((1, n), jnp.int32),
    )(xp, et, x2.reshape(nblocks, 1, _BN), e2.reshape(1, 1, k), iota)
    return idx


def _sc_gather(table, indices):
    n = indices.shape[1]
    d = table.shape[1]

    @pl.kernel(
        out_type=jax.ShapeDtypeStruct((n, d), table.dtype),
        mesh=plsc.VectorSubcoreMesh(core_axis_name="core",
                                    subcore_axis_name="subcore"),
    )
    def gather_kernel(tab_hbm, i_hbm, o_hbm):
        def body(i_vmem, o_vmem):
            pltpu.sync_copy(tab_hbm.at[i_vmem.at[0]], o_vmem)

        pltpu.emit_pipeline(
            body,
            grid=(n // _GATHER_WINDOW,),
            in_specs=[pl.BlockSpec((1, _GATHER_WINDOW),
                                   index_map=lambda i: (0, i))],
            out_specs=[pl.BlockSpec((_GATHER_WINDOW, d),
                                    index_map=lambda i: (i, 0))],
            core_axis_name=("core", "subcore"),
            dimension_semantics=(pltpu.PARALLEL,),
        )(i_hbm, o_hbm)

    return gather_kernel(table, indices)


def kernel(x, embeddings):
    b, c, h, w = x.shape
    xp = jnp.transpose(x, (0, 2, 3, 1)).reshape(b * h * w, c)
    x2 = jnp.sum(xp * xp, axis=-1)
    e2 = jnp.sum(embeddings * embeddings, axis=-1)
    flat_idx = _argmin_indices(xp, embeddings.T, x2, e2)
    k, d = embeddings.shape
    table = jnp.pad(embeddings, ((0, 0), (0, 128 - d)))
    xq = _sc_gather(table, flat_idx)[:, :d]
    xq = xq.reshape(b, h, w, c)
    xq = jnp.transpose(xq, (0, 3, 1, 2))
    return jax.lax.stop_gradient(xq - x) + x
